# Initial kernel scaffold; baseline (speedup 1.0000x reference)
import math

import jax
import jax.numpy as jnp
from jax import lax
from jax.experimental import pallas as pl
from jax.experimental.pallas import tpu as pltpu

D = 256
NQ = 300
NH = 8
HD = D // NH
NL = 2
NM = 4096
AK = 96
CK = 1024
NPF = 64
_NEG = -1e30

_LAYER_KEYS = (
    "sa_in_w", "sa_in_b", "sa_out_w", "sa_out_b",
    "ca_in_w", "ca_in_b", "ca_out_w", "ca_out_b",
    "align_q", "align_m", "ag1_w", "ag1_b", "ag2_w", "ag2_b",
    "cq", "cm", "ffn1_w", "ffn1_b", "ffn2_w", "ffn2_b",
    "n1_g", "n1_b", "n2_g", "n2_b", "n3_g", "n3_b",
)


def _gelu(x):
    return jax.nn.gelu(x, approximate=False)


def _lin(x, w):
    return lax.dot_general(x, w, (((1,), (1,)), ((), ())),
                           preferred_element_type=jnp.float32)


def _dot(a, b):
    return lax.dot_general(a, b, (((1,), (0,)), ((), ())),
                           preferred_element_type=jnp.float32)


def _normalize(x):
    n = jnp.sqrt(jnp.sum(x * x, axis=-1, keepdims=True))
    return x / jnp.maximum(n, 1e-6)


def _layernorm(x, g, b):
    m = jnp.mean(x, axis=-1, keepdims=True)
    v = jnp.mean((x - m) ** 2, axis=-1, keepdims=True)
    return (x - m) / jnp.sqrt(v + 1e-5) * g + b


def _row_kth_threshold(s, k, iters=40):
    r = s.shape[0]
    lo = jnp.full((r, 1), -1.02, jnp.float32)
    hi = jnp.full((r, 1), 1.02, jnp.float32)

    def body(_, carry):
        lo, hi = carry
        mid = 0.5 * (lo + hi)
        cnt = jnp.sum((s >= mid).astype(jnp.float32), axis=1, keepdims=True)
        ge = cnt >= k
        return jnp.where(ge, mid, lo), jnp.where(ge, hi, mid)

    lo, hi = lax.fori_loop(0, iters, body, (lo, hi))
    return lo


def _col_kth_threshold(s, k, iters=40):
    lo = jnp.full((1, 1), -1.02, jnp.float32)
    hi = jnp.full((1, 1), 1.02, jnp.float32)

    def body(_, carry):
        lo, hi = carry
        mid = 0.5 * (lo + hi)
        cnt = jnp.sum((s >= mid).astype(jnp.float32), keepdims=True)
        ge = cnt >= k
        return jnp.where(ge, mid, lo), jnp.where(ge, hi, mid)

    lo, hi = lax.fori_loop(0, iters, body, (lo, hi))
    return lo


def _masked_softmax(s, neg_add):
    s = s + neg_add
    mx = jnp.max(s, axis=1, keepdims=True)
    e = jnp.exp(s - mx)
    return e / jnp.sum(e, axis=1, keepdims=True)


def _decoder_kernel(names, *refs):
    w = dict(zip(names, refs[:-2]))
    out_ref, g_scr = refs[-2], refs[-1]

    mem = w["memory"][0]
    qf = w["query_feat"][...]
    qp = w["query_pos"][...]

    q_mean = jnp.mean(qf, axis=0, keepdims=True)
    qm = _lin(q_mean, w["bato_q_proj"][...])
    mp = _lin(mem, w["bato_m_proj"][...])
    aff = _lin(mp, qm)
    g1 = _gelu(_lin(mem, w["bato_g1_w"][...]) + w["bato_g1_b"][...])
    g2 = jax.nn.sigmoid(_lin(g1, w["bato_g2_w"][...]) + w["bato_g2_b"][...])
    memb = mem + jax.nn.sigmoid(aff) * g2 * mem

    xc = w["rpx_c"][0]
    yc = w["rpy_c"][0]
    xr = w["rpx_r"][0]
    yr = w["rpy_r"][0]
    dx = xc - xr
    dy = yc - yr
    dist = jnp.sqrt(dx * dx + dy * dy + 1e-8)
    logd = jnp.log(dist)
    ang = jnp.arctan2(dy, dx) * (1.0 / math.pi)
    feats = (dx, dy, logd, ang)
    for c in range(NPF // 8):
        sl = slice(c * 8, (c + 1) * 8)
        hid = w["pr_b1"][sl]
        for i in range(4):
            hid = hid + w["pr_w1"][i, sl] * feats[i][None]
        g_scr[sl] = _gelu(hid)

    def acc_body(j, a):
        return a + w["pr_w2t"][j] * g_scr[j][None]
    bias = lax.fori_loop(0, NPF, acc_body,
                         jnp.zeros((NH, NQ, NQ), jnp.float32))
    bias = bias + w["pr_b2"][...]

    q = qf
    for l in range(NL):
        lw = {k: w["L_" + k][l] for k in _LAYER_KEYS}

        qn = _normalize(_lin(q, lw["align_q"]))
        mn = _normalize(_lin(memb, lw["align_m"]))
        sim = _lin(qn, mn)
        thr = _row_kth_threshold(sim, AK)
        wts = _masked_softmax(sim, jnp.where(sim >= thr, 0.0, _NEG))
        aligned = _dot(wts, memb)
        h = _gelu(_lin(q, lw["ag1_w"][:, :D]) +
                  _lin(aligned, lw["ag1_w"][:, D:]) + lw["ag1_b"])
        gate = jax.nn.sigmoid(_lin(h, lw["ag2_w"]) + lw["ag2_b"])
        q = q + gate * aligned

        qpp = q + qp
        Q = _lin(qpp, lw["sa_in_w"][0:D]) + lw["sa_in_b"][:, 0:D]
        K = _lin(qpp, lw["sa_in_w"][D:2 * D]) + lw["sa_in_b"][:, D:2 * D]
        V = _lin(q, lw["sa_in_w"][2 * D:]) + lw["sa_in_b"][:, 2 * D:]
        scale = 1.0 / math.sqrt(HD)
        outs = []
        for hh in range(NH):
            hs = slice(hh * HD, (hh + 1) * HD)
            s = _lin(Q[:, hs], K[:, hs]) * scale + bias[hh]
            p = _masked_softmax(s, 0.0)
            outs.append(_dot(p, V[:, hs]))
        o = jnp.concatenate(outs, axis=1)
        q = _layernorm(q + _lin(o, lw["sa_out_w"]) + lw["sa_out_b"],
                       lw["n1_g"], lw["n1_b"])

        qn2 = _normalize(_lin(q, lw["cq"]))
        mn2 = _normalize(_lin(memb, lw["cm"]))
        sim2 = _lin(qn2, mn2)
        imp = jnp.max(sim2, axis=0, keepdims=True)
        thr2 = _col_kth_threshold(imp, CK)
        keyneg = jnp.where(imp >= thr2, 0.0, _NEG)

        Q2 = _lin(q + qp, lw["ca_in_w"][0:D]) + lw["ca_in_b"][:, 0:D]
        K2 = _lin(memb, lw["ca_in_w"][D:2 * D]) + lw["ca_in_b"][:, D:2 * D]
        V2 = _lin(memb, lw["ca_in_w"][2 * D:]) + lw["ca_in_b"][:, 2 * D:]
        outs = []
        for hh in range(NH):
            hs = slice(hh * HD, (hh + 1) * HD)
            s = _lin(Q2[:, hs], K2[:, hs]) * scale
            p = _masked_softmax(s, keyneg)
            outs.append(_dot(p, V2[:, hs]))
        o = jnp.concatenate(outs, axis=1)
        q = _layernorm(q + _lin(o, lw["ca_out_w"]) + lw["ca_out_b"],
                       lw["n2_g"], lw["n2_b"])

        ff = _lin(_gelu(_lin(q, lw["ffn1_w"]) + lw["ffn1_b"]), lw["ffn2_w"])
        q = _layernorm(q + ff + lw["ffn2_b"], lw["n3_g"], lw["n3_b"])

    out_ref[0] = q


def kernel(memory, ref_points, params):
    f32 = jnp.float32
    Bb = memory.shape[0]
    row = lambda a: a.reshape(1, -1).astype(f32)

    inputs = {
        "memory": memory.astype(f32),
        "rpx_c": ref_points[:, :, 0:1].astype(f32),
        "rpy_c": ref_points[:, :, 1:2].astype(f32),
        "rpx_r": ref_points[:, :, 0:1].transpose(0, 2, 1).astype(f32),
        "rpy_r": ref_points[:, :, 1:2].transpose(0, 2, 1).astype(f32),
        "query_feat": params["query_feat"].astype(f32),
        "query_pos": params["query_pos"].astype(f32),
        "bato_q_proj": params["bato"]["q_proj"].astype(f32),
        "bato_m_proj": params["bato"]["m_proj"].astype(f32),
        "bato_g1_w": params["bato"]["g1_w"].astype(f32),
        "bato_g1_b": row(params["bato"]["g1_b"]),
        "bato_g2_w": params["bato"]["g2_w"].astype(f32),
        "bato_g2_b": row(params["bato"]["g2_b"]),
        "pr_w1": params["posrel"]["w1"].T.reshape(4, NPF, 1, 1).astype(f32),
        "pr_b1": params["posrel"]["b1"].reshape(NPF, 1, 1).astype(f32),
        "pr_w2t": params["posrel"]["w2"].T.reshape(NPF, NH, 1, 1).astype(f32),
        "pr_b2": params["posrel"]["b2"].reshape(NH, 1, 1).astype(f32),
    }
    for k in _LAYER_KEYS:
        stk = jnp.stack([params["layers"][l][k] for l in range(NL)])
        if stk.ndim == 2:
            stk = stk[:, None, :]
        inputs["L_" + k] = stk.astype(f32)

    names = list(inputs.keys())
    arrays = [inputs[k] for k in names]

    def bspec(a, k):
        nd = a.ndim
        if k in ("memory", "rpx_c", "rpy_c", "rpx_r", "rpy_r"):
            blk = (1,) + a.shape[1:]
            return pl.BlockSpec(blk, lambda b, _n=nd: (b,) + (0,) * (_n - 1))
        return pl.BlockSpec(a.shape, lambda b, _n=nd: (0,) * _n)

    out = pl.pallas_call(
        lambda *refs: _decoder_kernel(names, *refs),
        grid=(Bb,),
        in_specs=[bspec(a, k) for k, a in zip(names, arrays)],
        out_specs=pl.BlockSpec((1, NQ, D), lambda b: (b, 0, 0)),
        out_shape=jax.ShapeDtypeStruct((Bb, NQ, D), f32),
        scratch_shapes=[pltpu.VMEM((NPF, NQ, NQ), f32)],
    )(*arrays)
    return out


# speedup vs baseline: 10.0472x; 10.0472x over previous
import math

import jax
import jax.numpy as jnp
from jax import lax
from jax.experimental import pallas as pl
from jax.experimental.pallas import tpu as pltpu

D = 256
NQ = 300
NH = 8
HD = D // NH
NL = 2
NM = 4096
AK = 96
CK = 1024
NPF = 64
_NEG = -1e30

_LAYER_KEYS = (
    "sa_in_w", "sa_in_b", "sa_out_w", "sa_out_b",
    "ca_in_w", "ca_in_b", "ca_out_w", "ca_out_b",
    "align_q", "align_m", "ag1_w", "ag1_b", "ag2_w", "ag2_b",
    "cq", "cm", "ffn1_w", "ffn1_b", "ffn2_w", "ffn2_b",
    "n1_g", "n1_b", "n2_g", "n2_b", "n3_g", "n3_b",
)


def _gelu(x):
    return 0.5 * x * (1.0 + lax.erf(x * (1.0 / math.sqrt(2.0))))


_BF = jnp.bfloat16


def _lin(x, w):
    return lax.dot_general(x.astype(_BF), w.astype(_BF),
                           (((1,), (1,)), ((), ())),
                           preferred_element_type=jnp.float32)


def _dot(a, b):
    return lax.dot_general(a.astype(_BF), b.astype(_BF),
                           (((1,), (0,)), ((), ())),
                           preferred_element_type=jnp.float32)


def _rowdot(a, b):
    p = a.astype(_BF).astype(jnp.float32) * b.astype(_BF).astype(jnp.float32)
    return jnp.sum(p, axis=1, keepdims=True)


def _normalize(x):
    n = jnp.sqrt(jnp.sum(x * x, axis=-1, keepdims=True))
    return x / jnp.maximum(n, 1e-6)


def _layernorm(x, g, b):
    m = jnp.mean(x, axis=-1, keepdims=True)
    v = jnp.mean((x - m) ** 2, axis=-1, keepdims=True)
    return (x - m) / jnp.sqrt(v + 1e-5) * g + b


def _row_kth_threshold(s, k, iters=40):
    r = s.shape[0]
    lo = jnp.full((r, 1), -1.02, jnp.float32)
    hi = jnp.full((r, 1), 1.02, jnp.float32)

    def body(_, carry):
        lo, hi = carry
        mid = 0.5 * (lo + hi)
        cnt = jnp.sum((s >= mid).astype(jnp.float32), axis=1, keepdims=True)
        ge = cnt >= k
        return jnp.where(ge, mid, lo), jnp.where(ge, hi, mid)

    lo, hi = lax.fori_loop(0, iters, body, (lo, hi))
    return lo


def _col_kth_threshold(s, k, iters=40):
    lo = jnp.full((1, 1), -1.02, jnp.float32)
    hi = jnp.full((1, 1), 1.02, jnp.float32)

    def body(_, carry):
        lo, hi = carry
        mid = 0.5 * (lo + hi)
        cnt = jnp.sum((s >= mid).astype(jnp.float32), keepdims=True)
        ge = cnt >= k
        return jnp.where(ge, mid, lo), jnp.where(ge, hi, mid)

    lo, hi = lax.fori_loop(0, iters, body, (lo, hi))
    return lo


def _masked_softmax(s, neg_add):
    s = s + neg_add
    mx = jnp.max(s, axis=1, keepdims=True)
    e = jnp.exp(s - mx)
    return e / jnp.sum(e, axis=1, keepdims=True)


def _decoder_kernel(names, *refs):
    w = dict(zip(names, refs[:-1]))
    out_ref = refs[-1]

    mem = w["memory"][0]
    qf = w["query_feat"][...]
    qp = w["query_pos"][...]

    q_mean = jnp.mean(qf, axis=0, keepdims=True)
    qm = _lin(q_mean, w["bato_q_proj"][...])
    mp = _lin(mem, w["bato_m_proj"][...])
    aff = _rowdot(mp, qm)
    g1 = _gelu(_lin(mem, w["bato_g1_w"][...]) + w["bato_g1_b"][...])
    g2 = jax.nn.sigmoid(_rowdot(g1, w["bato_g2_w"][...]) + w["bato_g2_b"][0])
    memb = mem + jax.nn.sigmoid(aff) * g2 * mem

    xc = w["rpx_c"][0]
    yc = w["rpy_c"][0]
    xr = w["rpx_r"][0]
    yr = w["rpy_r"][0]
    dx = xc - xr
    dy = yc - yr
    dist = jnp.sqrt(dx * dx + dy * dy + 1e-8)
    logd = jnp.log(dist)
    ang = jnp.arctan2(dy, dx) * (1.0 / math.pi)
    w1s, b1s, w2s, b2s = w["pr_w1"], w["pr_b1"], w["pr_w2t"], w["pr_b2"]

    def acc_body(j, acc):
        plane = (w1s[j, 0] * dx + w1s[j, 1] * dy +
                 w1s[j, 2] * logd + w1s[j, 3] * ang + b1s[j])
        g = _gelu(plane)
        return tuple(acc[hh] + w2s[j, hh] * g for hh in range(NH))

    zero = jnp.zeros((NQ, NQ), jnp.float32)
    acc = lax.fori_loop(0, NPF, acc_body, (zero,) * NH)
    bias = [acc[hh] + b2s[hh] for hh in range(NH)]

    q = qf
    for l in range(NL):
        lw = {k: w["L_" + k][l] for k in _LAYER_KEYS}

        qn = _normalize(_lin(q, lw["align_q"]))
        mn = _normalize(_lin(memb, lw["align_m"]))
        sim = _lin(qn, mn)
        thr = _row_kth_threshold(sim, AK)
        wts = _masked_softmax(sim, jnp.where(sim >= thr, 0.0, _NEG))
        aligned = _dot(wts, memb)
        h = _gelu(_lin(q, lw["ag1_w"][:, :D]) +
                  _lin(aligned, lw["ag1_w"][:, D:]) + lw["ag1_b"])
        gate = jax.nn.sigmoid(_lin(h, lw["ag2_w"]) + lw["ag2_b"])
        q = q + gate * aligned

        qpp = q + qp
        Q = _lin(qpp, lw["sa_in_w"][0:D]) + lw["sa_in_b"][:, 0:D]
        K = _lin(qpp, lw["sa_in_w"][D:2 * D]) + lw["sa_in_b"][:, D:2 * D]
        V = _lin(q, lw["sa_in_w"][2 * D:]) + lw["sa_in_b"][:, 2 * D:]
        scale = 1.0 / math.sqrt(HD)
        outs = []
        for hh in range(NH):
            hs = slice(hh * HD, (hh + 1) * HD)
            s = _lin(Q[:, hs], K[:, hs]) * scale + bias[hh]
            p = _masked_softmax(s, 0.0)
            outs.append(_dot(p, V[:, hs]))
        o = jnp.concatenate(outs, axis=1)
        q = _layernorm(q + _lin(o, lw["sa_out_w"]) + lw["sa_out_b"],
                       lw["n1_g"], lw["n1_b"])

        qn2 = _normalize(_lin(q, lw["cq"]))
        mn2 = _normalize(_lin(memb, lw["cm"]))
        sim2 = _lin(qn2, mn2)
        imp = jnp.max(sim2, axis=0, keepdims=True)
        thr2 = _col_kth_threshold(imp, CK)
        keyneg = jnp.where(imp >= thr2, 0.0, _NEG)

        Q2 = _lin(q + qp, lw["ca_in_w"][0:D]) + lw["ca_in_b"][:, 0:D]
        K2 = _lin(memb, lw["ca_in_w"][D:2 * D]) + lw["ca_in_b"][:, D:2 * D]
        V2 = _lin(memb, lw["ca_in_w"][2 * D:]) + lw["ca_in_b"][:, 2 * D:]
        outs = []
        for hh in range(NH):
            hs = slice(hh * HD, (hh + 1) * HD)
            s = _lin(Q2[:, hs], K2[:, hs]) * scale
            p = _masked_softmax(s, keyneg)
            outs.append(_dot(p, V2[:, hs]))
        o = jnp.concatenate(outs, axis=1)
        q = _layernorm(q + _lin(o, lw["ca_out_w"]) + lw["ca_out_b"],
                       lw["n2_g"], lw["n2_b"])

        ff = _lin(_gelu(_lin(q, lw["ffn1_w"]) + lw["ffn1_b"]), lw["ffn2_w"])
        q = _layernorm(q + ff + lw["ffn2_b"], lw["n3_g"], lw["n3_b"])

    out_ref[0] = q


def kernel(memory, ref_points, params):
    f32 = jnp.float32
    Bb = memory.shape[0]
    row = lambda a: a.reshape(1, -1).astype(f32)

    inputs = {
        "memory": memory.astype(f32),
        "rpx_c": ref_points[:, :, 0:1].astype(f32),
        "rpy_c": ref_points[:, :, 1:2].astype(f32),
        "rpx_r": ref_points[:, :, 0:1].transpose(0, 2, 1).astype(f32),
        "rpy_r": ref_points[:, :, 1:2].transpose(0, 2, 1).astype(f32),
        "query_feat": params["query_feat"].astype(f32),
        "query_pos": params["query_pos"].astype(f32),
        "bato_q_proj": params["bato"]["q_proj"].astype(f32),
        "bato_m_proj": params["bato"]["m_proj"].astype(f32),
        "bato_g1_w": params["bato"]["g1_w"].astype(f32),
        "bato_g1_b": row(params["bato"]["g1_b"]),
        "bato_g2_w": params["bato"]["g2_w"].astype(f32),
        "bato_g2_b": params["bato"]["g2_b"].reshape(1).astype(f32),
        "pr_w1": params["posrel"]["w1"].astype(f32),
        "pr_b1": params["posrel"]["b1"].reshape(NPF).astype(f32),
        "pr_w2t": params["posrel"]["w2"].T.astype(f32),
        "pr_b2": params["posrel"]["b2"].reshape(NH).astype(f32),
    }
    for k in _LAYER_KEYS:
        stk = jnp.stack([params["layers"][l][k] for l in range(NL)])
        if stk.ndim == 2:
            stk = stk[:, None, :]
        inputs["L_" + k] = stk.astype(f32)

    names = list(inputs.keys())
    arrays = [inputs[k] for k in names]

    _SMEM = ("pr_w1", "pr_b1", "pr_w2t", "pr_b2", "bato_g2_b")

    def bspec(a, k):
        nd = a.ndim
        if k in ("memory", "rpx_c", "rpy_c", "rpx_r", "rpy_r"):
            blk = (1,) + a.shape[1:]
            return pl.BlockSpec(blk, lambda b, _n=nd: (b,) + (0,) * (_n - 1))
        if k in _SMEM:
            return pl.BlockSpec(a.shape, lambda b, _n=nd: (0,) * _n,
                                memory_space=pltpu.SMEM)
        return pl.BlockSpec(a.shape, lambda b, _n=nd: (0,) * _n)

    out = pl.pallas_call(
        lambda *refs: _decoder_kernel(names, *refs),
        grid=(Bb,),
        in_specs=[bspec(a, k) for k, a in zip(names, arrays)],
        out_specs=pl.BlockSpec((1, NQ, D), lambda b: (b, 0, 0)),
        out_shape=jax.ShapeDtypeStruct((Bb, NQ, D), f32),
    )(*arrays)
    return out


# speedup vs baseline: 10.9191x; 1.0868x over previous
import math

import jax
import jax.numpy as jnp
from jax import lax
from jax.experimental import pallas as pl
from jax.experimental.pallas import tpu as pltpu

D = 256
NQ = 300
NH = 8
HD = D // NH
NL = 2
NM = 4096
AK = 96
CK = 1024
NPF = 64
_NEG = -1e30

_LAYER_KEYS = (
    "sa_in_w", "sa_in_b", "sa_out_w", "sa_out_b",
    "ca_in_w", "ca_in_b", "ca_out_w", "ca_out_b",
    "align_q", "align_m", "ag1_w", "ag1_b", "ag2_w", "ag2_b",
    "cq", "cm", "ffn1_w", "ffn1_b", "ffn2_w", "ffn2_b",
    "n1_g", "n1_b", "n2_g", "n2_b", "n3_g", "n3_b",
)


def _gelu(x):
    return 0.5 * x * (1.0 + lax.erf(x * (1.0 / math.sqrt(2.0))))


_BF = jnp.bfloat16


def _lin(x, w):
    return lax.dot_general(x.astype(_BF), w.astype(_BF),
                           (((1,), (1,)), ((), ())),
                           preferred_element_type=jnp.float32)


def _dot(a, b):
    return lax.dot_general(a.astype(_BF), b.astype(_BF),
                           (((1,), (0,)), ((), ())),
                           preferred_element_type=jnp.float32)


def _rowdot(a, b):
    p = a.astype(_BF).astype(jnp.float32) * b.astype(_BF).astype(jnp.float32)
    return jnp.sum(p, axis=1, keepdims=True)


def _normalize(x):
    n = jnp.sqrt(jnp.sum(x * x, axis=-1, keepdims=True))
    return x / jnp.maximum(n, 1e-6)


def _layernorm(x, g, b):
    m = jnp.mean(x, axis=-1, keepdims=True)
    v = jnp.mean((x - m) ** 2, axis=-1, keepdims=True)
    return (x - m) / jnp.sqrt(v + 1e-5) * g + b


def _row_kth_threshold(s, k, iters=40):
    lo = jnp.min(s, axis=1, keepdims=True) - 1e-3
    hi = jnp.max(s, axis=1, keepdims=True) + 1e-3
    cl = jnp.full(lo.shape, float(s.shape[1]), jnp.float32)

    def cond(carry):
        it, _, _, cl = carry
        return jnp.logical_and(it < iters, jnp.any(cl > k))

    def body(carry):
        it, lo, hi, cl = carry
        mid = 0.5 * (lo + hi)
        cnt = jnp.sum((s >= mid).astype(jnp.float32), axis=1, keepdims=True)
        ge = cnt >= k
        return (it + 1, jnp.where(ge, mid, lo), jnp.where(ge, hi, mid),
                jnp.where(ge, cnt, cl))

    _, lo, _, _ = lax.while_loop(cond, body, (0, lo, hi, cl))
    return lo


def _col_kth_threshold(s, k, iters=40):
    lo = jnp.full((1, 1), -1.02, jnp.float32)
    hi = jnp.full((1, 1), 1.02, jnp.float32)

    def body(_, carry):
        lo, hi = carry
        mid = 0.5 * (lo + hi)
        cnt = jnp.sum((s >= mid).astype(jnp.float32), keepdims=True)
        ge = cnt >= k
        return jnp.where(ge, mid, lo), jnp.where(ge, hi, mid)

    lo, hi = lax.fori_loop(0, iters, body, (lo, hi))
    return lo


def _masked_softmax(s, neg_add):
    s = s + neg_add
    mx = jnp.max(s, axis=1, keepdims=True)
    e = jnp.exp(s - mx)
    return e / jnp.sum(e, axis=1, keepdims=True)


def _decoder_kernel(names, *refs):
    w = dict(zip(names, refs[:-1]))
    out_ref = refs[-1]

    mem = w["memory"][0]
    qf = w["query_feat"][...]
    qp = w["query_pos"][...]

    q_mean = jnp.mean(qf, axis=0, keepdims=True)
    qm = _lin(q_mean, w["bato_q_proj"][...])
    mp = _lin(mem, w["bato_m_proj"][...])
    aff = _rowdot(mp, qm)
    g1 = _gelu(_lin(mem, w["bato_g1_w"][...]) + w["bato_g1_b"][...])
    g2 = jax.nn.sigmoid(_rowdot(g1, w["bato_g2_w"][...]) + w["bato_g2_b"][0])
    memb = mem + jax.nn.sigmoid(aff) * g2 * mem

    xc = w["rpx_c"][0]
    yc = w["rpy_c"][0]
    xr = w["rpx_r"][0]
    yr = w["rpy_r"][0]
    dx = xc - xr
    dy = yc - yr
    dist = jnp.sqrt(dx * dx + dy * dy + 1e-8)
    logd = jnp.log(dist)
    ang = jnp.arctan2(dy, dx) * (1.0 / math.pi)
    w1s, b1s, w2s, b2s = w["pr_w1"], w["pr_b1"], w["pr_w2t"], w["pr_b2"]

    def acc_body(j, acc):
        plane = (w1s[j, 0] * dx + w1s[j, 1] * dy +
                 w1s[j, 2] * logd + w1s[j, 3] * ang + b1s[j])
        g = _gelu(plane)
        return tuple(acc[hh] + w2s[j, hh] * g for hh in range(NH))

    zero = jnp.zeros((NQ, NQ), jnp.float32)
    acc = lax.fori_loop(0, NPF, acc_body, (zero,) * NH)
    bias = [acc[hh] + b2s[hh] for hh in range(NH)]

    q = qf
    for l in range(NL):
        lw = {k: w["L_" + k][l] for k in _LAYER_KEYS}

        qn = _normalize(_lin(q, lw["align_q"]))
        mn = _normalize(_lin(memb, lw["align_m"]))
        sim = _lin(qn, mn)
        thr = _row_kth_threshold(sim, AK)
        wts = _masked_softmax(sim, jnp.where(sim >= thr, 0.0, _NEG))
        aligned = _dot(wts, memb)
        h = _gelu(_lin(q, lw["ag1_w"][:, :D]) +
                  _lin(aligned, lw["ag1_w"][:, D:]) + lw["ag1_b"])
        gate = jax.nn.sigmoid(_lin(h, lw["ag2_w"]) + lw["ag2_b"])
        q = q + gate * aligned

        qpp = q + qp
        Q = _lin(qpp, lw["sa_in_w"][0:D]) + lw["sa_in_b"][:, 0:D]
        K = _lin(qpp, lw["sa_in_w"][D:2 * D]) + lw["sa_in_b"][:, D:2 * D]
        V = _lin(q, lw["sa_in_w"][2 * D:]) + lw["sa_in_b"][:, 2 * D:]
        scale = 1.0 / math.sqrt(HD)
        outs = []
        for hh in range(NH):
            hs = slice(hh * HD, (hh + 1) * HD)
            s = _lin(Q[:, hs], K[:, hs]) * scale + bias[hh]
            p = _masked_softmax(s, 0.0)
            outs.append(_dot(p, V[:, hs]))
        o = jnp.concatenate(outs, axis=1)
        q = _layernorm(q + _lin(o, lw["sa_out_w"]) + lw["sa_out_b"],
                       lw["n1_g"], lw["n1_b"])

        qn2 = _normalize(_lin(q, lw["cq"]))
        mn2 = _normalize(_lin(memb, lw["cm"]))
        sim2 = _lin(qn2, mn2)
        imp = jnp.max(sim2, axis=0, keepdims=True)
        thr2 = _col_kth_threshold(imp, CK)
        keyneg = jnp.where(imp >= thr2, 0.0, _NEG)

        Q2 = _lin(q + qp, lw["ca_in_w"][0:D]) + lw["ca_in_b"][:, 0:D]
        K2 = _lin(memb, lw["ca_in_w"][D:2 * D]) + lw["ca_in_b"][:, D:2 * D]
        V2 = _lin(memb, lw["ca_in_w"][2 * D:]) + lw["ca_in_b"][:, 2 * D:]
        outs = []
        for hh in range(NH):
            hs = slice(hh * HD, (hh + 1) * HD)
            s = _lin(Q2[:, hs], K2[:, hs]) * scale
            p = _masked_softmax(s, keyneg)
            outs.append(_dot(p, V2[:, hs]))
        o = jnp.concatenate(outs, axis=1)
        q = _layernorm(q + _lin(o, lw["ca_out_w"]) + lw["ca_out_b"],
                       lw["n2_g"], lw["n2_b"])

        ff = _lin(_gelu(_lin(q, lw["ffn1_w"]) + lw["ffn1_b"]), lw["ffn2_w"])
        q = _layernorm(q + ff + lw["ffn2_b"], lw["n3_g"], lw["n3_b"])

    out_ref[0] = q


def kernel(memory, ref_points, params):
    f32 = jnp.float32
    Bb = memory.shape[0]
    row = lambda a: a.reshape(1, -1).astype(f32)

    inputs = {
        "memory": memory.astype(f32),
        "rpx_c": ref_points[:, :, 0:1].astype(f32),
        "rpy_c": ref_points[:, :, 1:2].astype(f32),
        "rpx_r": ref_points[:, :, 0:1].transpose(0, 2, 1).astype(f32),
        "rpy_r": ref_points[:, :, 1:2].transpose(0, 2, 1).astype(f32),
        "query_feat": params["query_feat"].astype(f32),
        "query_pos": params["query_pos"].astype(f32),
        "bato_q_proj": params["bato"]["q_proj"].astype(f32),
        "bato_m_proj": params["bato"]["m_proj"].astype(f32),
        "bato_g1_w": params["bato"]["g1_w"].astype(f32),
        "bato_g1_b": row(params["bato"]["g1_b"]),
        "bato_g2_w": params["bato"]["g2_w"].astype(f32),
        "bato_g2_b": params["bato"]["g2_b"].reshape(1).astype(f32),
        "pr_w1": params["posrel"]["w1"].astype(f32),
        "pr_b1": params["posrel"]["b1"].reshape(NPF).astype(f32),
        "pr_w2t": params["posrel"]["w2"].T.astype(f32),
        "pr_b2": params["posrel"]["b2"].reshape(NH).astype(f32),
    }
    for k in _LAYER_KEYS:
        stk = jnp.stack([params["layers"][l][k] for l in range(NL)])
        if stk.ndim == 2:
            stk = stk[:, None, :]
        inputs["L_" + k] = stk.astype(f32)

    names = list(inputs.keys())
    arrays = [inputs[k] for k in names]

    _SMEM = ("pr_w1", "pr_b1", "pr_w2t", "pr_b2", "bato_g2_b")

    def bspec(a, k):
        nd = a.ndim
        if k in ("memory", "rpx_c", "rpy_c", "rpx_r", "rpy_r"):
            blk = (1,) + a.shape[1:]
            return pl.BlockSpec(blk, lambda b, _n=nd: (b,) + (0,) * (_n - 1))
        if k in _SMEM:
            return pl.BlockSpec(a.shape, lambda b, _n=nd: (0,) * _n,
                                memory_space=pltpu.SMEM)
        return pl.BlockSpec(a.shape, lambda b, _n=nd: (0,) * _n)

    out = pl.pallas_call(
        lambda *refs: _decoder_kernel(names, *refs),
        grid=(Bb,),
        in_specs=[bspec(a, k) for k, a in zip(names, arrays)],
        out_specs=pl.BlockSpec((1, NQ, D), lambda b: (b, 0, 0)),
        out_shape=jax.ShapeDtypeStruct((Bb, NQ, D), f32),
    )(*arrays)
    return out


# speedup vs baseline: 11.3005x; 1.0349x over previous
import functools
import math

import jax
import jax.numpy as jnp
from jax import lax
from jax.experimental import pallas as pl
from jax.experimental.pallas import tpu as pltpu
from jax.experimental.pallas import tpu_sc as plsc

D = 256
NQ = 300
NH = 8
HD = D // NH
NL = 2
NM = 4096
AK = 96
CK = 1024
NPF = 64
_NEG = -1e30

_LAYER_KEYS = (
    "sa_in_w", "sa_in_b", "sa_out_w", "sa_out_b",
    "ca_in_w", "ca_in_b", "ca_out_w", "ca_out_b",
    "align_q", "align_m", "ag1_w", "ag1_b", "ag2_w", "ag2_b",
    "cq", "cm", "ffn1_w", "ffn1_b", "ffn2_w", "ffn2_b",
    "n1_g", "n1_b", "n2_g", "n2_b", "n3_g", "n3_b",
)

_BF = jnp.bfloat16


def _gelu(x):
    return 0.5 * x * (1.0 + lax.erf(x * (1.0 / math.sqrt(2.0))))


def _lin(x, w):
    return lax.dot_general(x.astype(_BF), w.astype(_BF),
                           (((1,), (1,)), ((), ())),
                           preferred_element_type=jnp.float32)


def _dot(a, b):
    return lax.dot_general(a.astype(_BF), b.astype(_BF),
                           (((1,), (0,)), ((), ())),
                           preferred_element_type=jnp.float32)


def _rowdot(a, b):
    p = a.astype(_BF).astype(jnp.float32) * b.astype(_BF).astype(jnp.float32)
    return jnp.sum(p, axis=1, keepdims=True)


def _normalize(x):
    n = jnp.sqrt(jnp.sum(x * x, axis=-1, keepdims=True))
    return x / jnp.maximum(n, 1e-6)


def _layernorm(x, g, b):
    m = jnp.mean(x, axis=-1, keepdims=True)
    v = jnp.mean((x - m) ** 2, axis=-1, keepdims=True)
    return (x - m) / jnp.sqrt(v + 1e-5) * g + b


def _row_kth_threshold(s, k, iters=40):
    lo = jnp.min(s, axis=1, keepdims=True) - 1e-3
    hi = jnp.max(s, axis=1, keepdims=True) + 1e-3
    cl = jnp.full(lo.shape, float(s.shape[1]), jnp.float32)

    def cond(carry):
        it, _, _, cl = carry
        return jnp.logical_and(it < iters, jnp.any(cl > k))

    def body(carry):
        it, lo, hi, cl = carry
        mid = 0.5 * (lo + hi)
        cnt = jnp.sum((s >= mid).astype(jnp.float32), axis=1, keepdims=True)
        ge = cnt >= k
        return (it + 1, jnp.where(ge, mid, lo), jnp.where(ge, hi, mid),
                jnp.where(ge, cnt, cl))

    _, lo, _, _ = lax.while_loop(cond, body, (0, lo, hi, cl))
    return lo


def _masked_softmax(s, neg_add):
    s = s + neg_add
    mx = jnp.max(s, axis=1, keepdims=True)
    e = jnp.exp(s - mx)
    return e / jnp.sum(e, axis=1, keepdims=True)


def _bato(w, mem, qf):
    q_mean = jnp.mean(qf, axis=0, keepdims=True)
    qm = _lin(q_mean, w["bato_q_proj"][...])
    mp = _lin(mem, w["bato_m_proj"][...])
    aff = _rowdot(mp, qm)
    g1 = _gelu(_lin(mem, w["bato_g1_w"][...]) + w["bato_g1_b"][...])
    g2 = jax.nn.sigmoid(_rowdot(g1, w["bato_g2_w"][...]) + w["bato_g2_b"][0])
    return mem + jax.nn.sigmoid(aff) * g2 * mem


def _posrel(w):
    xc = w["rpx_c"][0]
    yc = w["rpy_c"][0]
    xr = w["rpx_r"][0]
    yr = w["rpy_r"][0]
    dx = xc - xr
    dy = yc - yr
    dist = jnp.sqrt(dx * dx + dy * dy + 1e-8)
    logd = jnp.log(dist)
    ang = jnp.arctan2(dy, dx) * (1.0 / math.pi)
    w1s, b1s, w2s, b2s = w["pr_w1"], w["pr_b1"], w["pr_w2t"], w["pr_b2"]

    def acc_body(j, acc):
        plane = (w1s[j, 0] * dx + w1s[j, 1] * dy +
                 w1s[j, 2] * logd + w1s[j, 3] * ang + b1s[j])
        g = _gelu(plane)
        return tuple(acc[hh] + w2s[j, hh] * g for hh in range(NH))

    zero = jnp.zeros((NQ, NQ), jnp.float32)
    acc = lax.fori_loop(0, NPF, acc_body, (zero,) * NH)
    return [acc[hh] + b2s[hh] for hh in range(NH)]


def _align(lw, q, memb):
    qn = _normalize(_lin(q, lw["align_q"]))
    mn = _normalize(_lin(memb, lw["align_m"]))
    sim = _lin(qn, mn)
    thr = _row_kth_threshold(sim, AK)
    wts = _masked_softmax(sim, jnp.where(sim >= thr, 0.0, _NEG))
    aligned = _dot(wts, memb)
    h = _gelu(_lin(q, lw["ag1_w"][:, :D]) +
              _lin(aligned, lw["ag1_w"][:, D:]) + lw["ag1_b"])
    gate = jax.nn.sigmoid(_lin(h, lw["ag2_w"]) + lw["ag2_b"])
    return q + gate * aligned


def _mha(q_in, kv, v_in, iw, ib, ow, ob, bias=None, keyneg=None):
    Q = _lin(q_in, iw[0:D]) + ib[:, 0:D]
    K = _lin(kv, iw[D:2 * D]) + ib[:, D:2 * D]
    V = _lin(v_in, iw[2 * D:]) + ib[:, 2 * D:]
    scale = 1.0 / math.sqrt(HD)
    outs = []
    for hh in range(NH):
        hs = slice(hh * HD, (hh + 1) * HD)
        s = _lin(Q[:, hs], K[:, hs]) * scale
        if bias is not None:
            s = s + bias[hh]
        p = _masked_softmax(s, 0.0 if keyneg is None else keyneg)
        outs.append(_dot(p, V[:, hs]))
    o = jnp.concatenate(outs, axis=1)
    return _lin(o, ow) + ob


def _select_rank_products(lw, q, memb):
    qn2 = _normalize(_lin(q, lw["cq"]))
    mn2 = _normalize(_lin(memb, lw["cm"]))
    sim2 = _lin(qn2, mn2)
    imp = jnp.max(sim2, axis=0, keepdims=True)

    lo = jnp.full((1, 1), -1.02, jnp.float32)
    hi = jnp.full((1, 1), 1.02, jnp.float32)

    def body(_, carry):
        lo, hi = carry
        mid = 0.5 * (lo + hi)
        cnt = jnp.sum((imp >= mid).astype(jnp.float32), keepdims=True)
        ge = cnt >= CK
        return jnp.where(ge, mid, lo), jnp.where(ge, hi, mid)

    lo, hi = lax.fori_loop(0, 40, body, (lo, hi))
    maskf = (imp >= lo).astype(jnp.float32)

    rank = maskf
    sft = 1
    while sft < NM:
        z = jnp.zeros((1, sft), jnp.float32)
        rank = rank + jnp.concatenate([z, rank[:, :NM - sft]], axis=1)
        sft *= 2
    sel = jnp.where(maskf > 0.0, rank, 0.0)

    jj = lax.broadcasted_iota(jnp.int32, (CK, NM), 0).astype(jnp.float32) + 1.0
    ind = (jj == sel).astype(_BF)
    li = lax.broadcasted_iota(jnp.int32, (8, NM), 1).astype(jnp.float32)
    ri = lax.broadcasted_iota(jnp.int32, (8, NM), 0).astype(jnp.float32)
    hi64 = jnp.floor(li * (1.0 / 64.0))
    pos2 = jnp.where(ri == 0.0, hi64,
                     jnp.where(ri == 1.0, li - 64.0 * hi64, 0.0))
    return lax.dot_general(ind, pos2.astype(_BF), (((1,), (1,)), ((), ())),
                           preferred_element_type=jnp.float32)


def _ffn(lw, q):
    ff = _lin(_gelu(_lin(q, lw["ffn1_w"]) + lw["ffn1_b"]), lw["ffn2_w"])
    return _layernorm(q + ff + lw["ffn2_b"], lw["n3_g"], lw["n3_b"])


def _stage1_kernel(names, *refs):
    w = dict(zip(names, refs[:-4]))
    q_ref, memb_ref, bias_ref, prod_ref = refs[-4:]

    mem = w["memory"][0]
    qf = w["query_feat"][...]
    qp = w["query_pos"][...]

    memb = _bato(w, mem, qf)
    bias = _posrel(w)

    lw = {k: w["L_" + k][0] for k in _LAYER_KEYS}
    q = _align(lw, qf, memb)
    q2 = _mha(q + qp, q + qp, q, lw["sa_in_w"], lw["sa_in_b"],
              lw["sa_out_w"], lw["sa_out_b"], bias=bias)
    q = _layernorm(q + q2, lw["n1_g"], lw["n1_b"])

    prod = _select_rank_products(lw, q, memb)

    q_ref[0] = q
    memb_ref[0] = memb
    for hh in range(NH):
        bias_ref[0, hh] = bias[hh]
    prod_ref[0] = prod


def _stage2_kernel(names, *refs):
    w = dict(zip(names, refs[:-2]))
    q_ref, prod_ref = refs[-2:]

    qp = w["query_pos"][...]
    q = w["q_in"][0]
    memb = w["memb_in"][0]
    msel = w["msel_in"][0]
    bias = [w["bias_in"][0, hh] for hh in range(NH)]

    lw = {k: w["L_" + k][0] for k in _LAYER_KEYS}
    q2 = _mha(q + qp, msel, msel, lw["ca_in_w"], lw["ca_in_b"],
              lw["ca_out_w"], lw["ca_out_b"])
    q = _layernorm(q + q2, lw["n2_g"], lw["n2_b"])
    q = _ffn(lw, q)

    lw = {k: w["L_" + k][1] for k in _LAYER_KEYS}
    q = _align(lw, q, memb)
    q2 = _mha(q + qp, q + qp, q, lw["sa_in_w"], lw["sa_in_b"],
              lw["sa_out_w"], lw["sa_out_b"], bias=bias)
    q = _layernorm(q + q2, lw["n1_g"], lw["n1_b"])

    prod = _select_rank_products(lw, q, memb)

    q_ref[0] = q
    prod_ref[0] = prod


def _stage3_kernel(names, *refs):
    w = dict(zip(names, refs[:-1]))
    out_ref = refs[-1]

    qp = w["query_pos"][...]
    q = w["q_in"][0]
    msel = w["msel_in"][0]

    lw = {k: w["L_" + k][1] for k in _LAYER_KEYS}
    q2 = _mha(q + qp, msel, msel, lw["ca_in_w"], lw["ca_in_b"],
              lw["ca_out_w"], lw["ca_out_b"])
    q = _layernorm(q + q2, lw["n2_g"], lw["n2_b"])
    out_ref[0] = _ffn(lw, q)


def _sc_gather(table, idx):
    NC, NS = 2, 16
    NW = NC * NS
    n, d = idx.shape[0], table.shape[1]
    bpw = n // NW
    mesh = plsc.VectorSubcoreMesh(core_axis_name="c", subcore_axis_name="s",
                                  num_cores=NC, num_subcores=NS)

    @functools.partial(
        pl.kernel, mesh=mesh,
        out_type=jax.ShapeDtypeStruct((n, d), jnp.float32),
        scratch_types=[
            pltpu.VMEM((bpw,), jnp.int32),
            pltpu.VMEM((bpw, d), jnp.float32),
            pltpu.SemaphoreType.DMA,
        ],
    )
    def k(table_hbm, idx_hbm, out_hbm, idx_v, rows_v, sem):
        wid = lax.axis_index("s") * NC + lax.axis_index("c")
        base = wid * bpw
        pltpu.sync_copy(idx_hbm.at[pl.ds(base, bpw)], idx_v)
        pltpu.async_copy(table_hbm.at[idx_v], rows_v, sem).wait()
        pltpu.sync_copy(rows_v, out_hbm.at[pl.ds(base, bpw)])

    return k(table, idx)


def _prod_to_flat_idx(prod, Bb):
    idx = (prod[..., 0] * 64.0 + prod[..., 1]).astype(jnp.int32)
    return (idx + (jnp.arange(Bb, dtype=jnp.int32) * NM)[:, None]).reshape(-1)


def kernel(memory, ref_points, params):
    f32 = jnp.float32
    Bb = memory.shape[0]

    base = {
        "memory": memory.astype(f32),
        "rpx_c": ref_points[:, :, 0:1].astype(f32),
        "rpy_c": ref_points[:, :, 1:2].astype(f32),
        "rpx_r": ref_points[:, :, 0:1].transpose(0, 2, 1).astype(f32),
        "rpy_r": ref_points[:, :, 1:2].transpose(0, 2, 1).astype(f32),
        "query_feat": params["query_feat"].astype(f32),
        "query_pos": params["query_pos"].astype(f32),
        "bato_q_proj": params["bato"]["q_proj"].astype(f32),
        "bato_m_proj": params["bato"]["m_proj"].astype(f32),
        "bato_g1_w": params["bato"]["g1_w"].astype(f32),
        "bato_g1_b": params["bato"]["g1_b"].reshape(1, -1).astype(f32),
        "bato_g2_w": params["bato"]["g2_w"].astype(f32),
        "bato_g2_b": params["bato"]["g2_b"].reshape(1).astype(f32),
        "pr_w1": params["posrel"]["w1"].astype(f32),
        "pr_b1": params["posrel"]["b1"].reshape(NPF).astype(f32),
        "pr_w2t": params["posrel"]["w2"].T.astype(f32),
        "pr_b2": params["posrel"]["b2"].reshape(NH).astype(f32),
    }
    for k in _LAYER_KEYS:
        stk = jnp.stack([params["layers"][l][k] for l in range(NL)])
        if stk.ndim == 2:
            stk = stk[:, None, :]
        base["L_" + k] = stk.astype(f32)

    _SMEM = ("pr_w1", "pr_b1", "pr_w2t", "pr_b2", "bato_g2_b")
    _BATCHED = ("memory", "rpx_c", "rpy_c", "rpx_r", "rpy_r",
                "q_in", "memb_in", "msel_in", "bias_in")

    def bspec(a, k):
        nd = a.ndim
        if k in _BATCHED:
            blk = (1,) + a.shape[1:]
            return pl.BlockSpec(blk, lambda b, _n=nd: (b,) + (0,) * (_n - 1))
        if k in _SMEM:
            return pl.BlockSpec(a.shape, lambda b, _n=nd: (0,) * _n,
                                memory_space=pltpu.SMEM)
        return pl.BlockSpec(a.shape, lambda b, _n=nd: (0,) * _n)

    def call(body, inputs, outs):
        names = list(inputs.keys())
        arrays = [inputs[k] for k in names]
        return pl.pallas_call(
            lambda *refs: body(names, *refs),
            grid=(Bb,),
            in_specs=[bspec(a, k) for k, a in zip(names, arrays)],
            out_specs=[pl.BlockSpec((1,) + s[1:],
                                    lambda b, _n=len(s): (b,) + (0,) * (_n - 1))
                       for s in outs],
            out_shape=[jax.ShapeDtypeStruct(s, f32) for s in outs],
        )(*arrays)

    q1, memb, bias, prod0 = call(
        _stage1_kernel, base,
        [(Bb, NQ, D), (Bb, NM, D), (Bb, NH, NQ, NQ), (Bb, CK, 8)])

    table = memb.reshape(Bb * NM, D)
    msel0 = _sc_gather(table, _prod_to_flat_idx(prod0, Bb))

    s2 = dict(base)
    s2.update(q_in=q1, memb_in=memb, bias_in=bias,
              msel_in=msel0.reshape(Bb, CK, D))
    q2, prod1 = call(_stage2_kernel, s2, [(Bb, NQ, D), (Bb, CK, 8)])

    msel1 = _sc_gather(table, _prod_to_flat_idx(prod1, Bb))

    s3 = dict(base)
    s3.update(q_in=q2, msel_in=msel1.reshape(Bb, CK, D))
    (out,) = call(_stage3_kernel, s3, [(Bb, NQ, D)])
    return out


# speedup vs baseline: 11.5288x; 1.0202x over previous
import functools
import math

import jax
import jax.numpy as jnp
from jax import lax
from jax.experimental import pallas as pl
from jax.experimental.pallas import tpu as pltpu
from jax.experimental.pallas import tpu_sc as plsc

D = 256
NQ = 300
NH = 8
HD = D // NH
NL = 2
NM = 4096
AK = 96
CK = 1024
NPF = 64
_NEG = -1e30

_LAYER_KEYS = (
    "sa_in_w", "sa_in_b", "sa_out_w", "sa_out_b",
    "ca_in_w", "ca_in_b", "ca_out_w", "ca_out_b",
    "align_q", "align_m", "ag1_w", "ag1_b", "ag2_w", "ag2_b",
    "cq", "cm", "ffn1_w", "ffn1_b", "ffn2_w", "ffn2_b",
    "n1_g", "n1_b", "n2_g", "n2_b", "n3_g", "n3_b",
)

_BF = jnp.bfloat16


def _gelu(x):
    return 0.5 * x * (1.0 + lax.erf(x * (1.0 / math.sqrt(2.0))))


def _lin(x, w):
    return lax.dot_general(x.astype(_BF), w.astype(_BF),
                           (((1,), (1,)), ((), ())),
                           preferred_element_type=jnp.float32)


def _dot(a, b):
    return lax.dot_general(a.astype(_BF), b.astype(_BF),
                           (((1,), (0,)), ((), ())),
                           preferred_element_type=jnp.float32)


def _rowdot(a, b):
    p = a.astype(_BF).astype(jnp.float32) * b.astype(_BF).astype(jnp.float32)
    return jnp.sum(p, axis=1, keepdims=True)


def _normalize(x):
    n = jnp.sqrt(jnp.sum(x * x, axis=-1, keepdims=True))
    return x / jnp.maximum(n, 1e-6)


def _layernorm(x, g, b):
    m = jnp.mean(x, axis=-1, keepdims=True)
    v = jnp.mean((x - m) ** 2, axis=-1, keepdims=True)
    return (x - m) / jnp.sqrt(v + 1e-5) * g + b


def _row_kth_threshold(s, k, iters=40):
    lo = jnp.min(s, axis=1, keepdims=True) - 1e-3
    hi = jnp.max(s, axis=1, keepdims=True) + 1e-3
    cl = jnp.full(lo.shape, float(s.shape[1]), jnp.float32)

    def cond(carry):
        it, _, _, cl = carry
        return jnp.logical_and(it < iters, jnp.any(cl > k))

    def body(carry):
        it, lo, hi, cl = carry
        mid = 0.5 * (lo + hi)
        cnt = jnp.sum((s >= mid).astype(jnp.float32), axis=1, keepdims=True)
        ge = cnt >= k
        return (it + 1, jnp.where(ge, mid, lo), jnp.where(ge, hi, mid),
                jnp.where(ge, cnt, cl))

    _, lo, _, _ = lax.while_loop(cond, body, (0, lo, hi, cl))
    return lo


def _masked_softmax(s, neg_add):
    s = s + neg_add
    mx = jnp.max(s, axis=1, keepdims=True)
    e = jnp.exp(s - mx)
    return e / jnp.sum(e, axis=1, keepdims=True)


def _bato(w, mem, qf):
    q_mean = jnp.mean(qf, axis=0, keepdims=True)
    qm = _lin(q_mean, w["bato_q_proj"][...])
    mp = _lin(mem, w["bato_m_proj"][...])
    aff = _rowdot(mp, qm)
    g1 = _gelu(_lin(mem, w["bato_g1_w"][...]) + w["bato_g1_b"][...])
    g2 = jax.nn.sigmoid(_rowdot(g1, w["bato_g2_w"][...]) + w["bato_g2_b"][0])
    return mem + jax.nn.sigmoid(aff) * g2 * mem


def _posrel(w):
    xc = w["rpx_c"][0]
    yc = w["rpy_c"][0]
    xr = w["rpx_r"][0]
    yr = w["rpy_r"][0]
    dx = xc - xr
    dy = yc - yr
    dist = jnp.sqrt(dx * dx + dy * dy + 1e-8)
    logd = jnp.log(dist)
    ang = jnp.arctan2(dy, dx) * (1.0 / math.pi)
    w1s, b1s, w2s, b2s = w["pr_w1"], w["pr_b1"], w["pr_w2t"], w["pr_b2"]

    def acc_body(j, acc):
        plane = (w1s[j, 0] * dx + w1s[j, 1] * dy +
                 w1s[j, 2] * logd + w1s[j, 3] * ang + b1s[j])
        g = _gelu(plane)
        return tuple(acc[hh] + w2s[j, hh] * g for hh in range(NH))

    zero = jnp.zeros((NQ, NQ), jnp.float32)
    acc = lax.fori_loop(0, NPF, acc_body, (zero,) * NH)
    return [acc[hh] + b2s[hh] for hh in range(NH)]


def _align(lw, q, memb):
    qn = _normalize(_lin(q, lw["align_q"]))
    mn = _normalize(_lin(memb, lw["align_m"]))
    sim = _lin(qn, mn)
    thr = _row_kth_threshold(sim, AK)
    wts = _masked_softmax(sim, jnp.where(sim >= thr, 0.0, _NEG))
    aligned = _dot(wts, memb)
    h = _gelu(_lin(q, lw["ag1_w"][:, :D]) +
              _lin(aligned, lw["ag1_w"][:, D:]) + lw["ag1_b"])
    gate = jax.nn.sigmoid(_lin(h, lw["ag2_w"]) + lw["ag2_b"])
    return q + gate * aligned


def _mha(q_in, kv, v_in, iw, ib, ow, ob, bias=None, keyneg=None):
    Q = _lin(q_in, iw[0:D]) + ib[:, 0:D]
    K = _lin(kv, iw[D:2 * D]) + ib[:, D:2 * D]
    V = _lin(v_in, iw[2 * D:]) + ib[:, 2 * D:]
    scale = 1.0 / math.sqrt(HD)
    outs = []
    for hh in range(NH):
        hs = slice(hh * HD, (hh + 1) * HD)
        s = _lin(Q[:, hs], K[:, hs]) * scale
        if bias is not None:
            s = s + bias[hh]
        p = _masked_softmax(s, 0.0 if keyneg is None else keyneg)
        outs.append(_dot(p, V[:, hs]))
    o = jnp.concatenate(outs, axis=1)
    return _lin(o, ow) + ob


def _select_rank_products(lw, q, memb):
    qn2 = _normalize(_lin(q, lw["cq"]))
    mn2 = _normalize(_lin(memb, lw["cm"]))
    sim2 = _lin(qn2, mn2)
    imp = jnp.max(sim2, axis=0, keepdims=True)

    lo = jnp.full((1, 1), -1.02, jnp.float32)
    hi = jnp.full((1, 1), 1.02, jnp.float32)

    def body(_, carry):
        lo, hi = carry
        mid = 0.5 * (lo + hi)
        cnt = jnp.sum((imp >= mid).astype(jnp.float32), keepdims=True)
        ge = cnt >= CK
        return jnp.where(ge, mid, lo), jnp.where(ge, hi, mid)

    lo, hi = lax.fori_loop(0, 40, body, (lo, hi))
    maskf = (imp >= lo).astype(jnp.float32)

    rank = maskf
    sft = 1
    while sft < NM:
        z = jnp.zeros((1, sft), jnp.float32)
        rank = rank + jnp.concatenate([z, rank[:, :NM - sft]], axis=1)
        sft *= 2
    sel = jnp.where(maskf > 0.0, rank, 0.0)

    jj = lax.broadcasted_iota(jnp.int32, (CK, NM), 0).astype(jnp.float32) + 1.0
    ind = (jj == sel).astype(_BF)
    li = lax.broadcasted_iota(jnp.int32, (8, NM), 1).astype(jnp.float32)
    ri = lax.broadcasted_iota(jnp.int32, (8, NM), 0).astype(jnp.float32)
    hi64 = jnp.floor(li * (1.0 / 64.0))
    pos2 = jnp.where(ri == 0.0, hi64,
                     jnp.where(ri == 1.0, li - 64.0 * hi64, 0.0))
    return lax.dot_general(ind, pos2.astype(_BF), (((1,), (1,)), ((), ())),
                           preferred_element_type=jnp.float32)


def _ffn(lw, q):
    ff = _lin(_gelu(_lin(q, lw["ffn1_w"]) + lw["ffn1_b"]), lw["ffn2_w"])
    return _layernorm(q + ff + lw["ffn2_b"], lw["n3_g"], lw["n3_b"])


def _stage1_kernel(names, *refs):
    w = dict(zip(names, refs[:-4]))
    q_ref, memb_ref, bias_ref, prod_ref = refs[-4:]

    mem = w["memory"][0]
    qf = w["query_feat"][...]
    qp = w["query_pos"][...]

    memb = _bato(w, mem, qf)
    bias = _posrel(w)

    lw = {k: w["L_" + k][0] for k in _LAYER_KEYS if "L_" + k in w}
    q = _align(lw, qf, memb)
    q2 = _mha(q + qp, q + qp, q, lw["sa_in_w"], lw["sa_in_b"],
              lw["sa_out_w"], lw["sa_out_b"], bias=bias)
    q = _layernorm(q + q2, lw["n1_g"], lw["n1_b"])

    prod = _select_rank_products(lw, q, memb)

    q_ref[0] = q
    memb_ref[0] = memb
    for hh in range(NH):
        bias_ref[0, hh] = bias[hh]
    prod_ref[0] = prod


def _stage2_kernel(names, *refs):
    w = dict(zip(names, refs[:-2]))
    q_ref, prod_ref = refs[-2:]

    qp = w["query_pos"][...]
    q = w["q_in"][0]
    memb = w["memb_in"][0]
    msel = w["msel_in"][0]
    bias = [w["bias_in"][0, hh] for hh in range(NH)]

    lw = {k: w["L_" + k][0] for k in _LAYER_KEYS if "L_" + k in w}
    q2 = _mha(q + qp, msel, msel, lw["ca_in_w"], lw["ca_in_b"],
              lw["ca_out_w"], lw["ca_out_b"])
    q = _layernorm(q + q2, lw["n2_g"], lw["n2_b"])
    q = _ffn(lw, q)

    lw = {k: w["L_" + k][1] for k in _LAYER_KEYS if "L_" + k in w}
    q = _align(lw, q, memb)
    q2 = _mha(q + qp, q + qp, q, lw["sa_in_w"], lw["sa_in_b"],
              lw["sa_out_w"], lw["sa_out_b"], bias=bias)
    q = _layernorm(q + q2, lw["n1_g"], lw["n1_b"])

    prod = _select_rank_products(lw, q, memb)

    q_ref[0] = q
    prod_ref[0] = prod


def _stage3_kernel(names, *refs):
    w = dict(zip(names, refs[:-1]))
    out_ref = refs[-1]

    qp = w["query_pos"][...]
    q = w["q_in"][0]
    msel = w["msel_in"][0]

    lw = {k: w["L_" + k][1] for k in _LAYER_KEYS if "L_" + k in w}
    q2 = _mha(q + qp, msel, msel, lw["ca_in_w"], lw["ca_in_b"],
              lw["ca_out_w"], lw["ca_out_b"])
    q = _layernorm(q + q2, lw["n2_g"], lw["n2_b"])
    out_ref[0] = _ffn(lw, q)


def _sc_gather(table, idx):
    NC, NS = 2, 16
    NW = NC * NS
    n, d = idx.shape[0], table.shape[1]
    bpw = n // NW
    mesh = plsc.VectorSubcoreMesh(core_axis_name="c", subcore_axis_name="s",
                                  num_cores=NC, num_subcores=NS)

    @functools.partial(
        pl.kernel, mesh=mesh,
        out_type=jax.ShapeDtypeStruct((n, d), jnp.float32),
        scratch_types=[
            pltpu.VMEM((bpw,), jnp.int32),
            pltpu.VMEM((bpw, d), jnp.float32),
            pltpu.SemaphoreType.DMA,
        ],
    )
    def k(table_hbm, idx_hbm, out_hbm, idx_v, rows_v, sem):
        wid = lax.axis_index("s") * NC + lax.axis_index("c")
        base = wid * bpw
        pltpu.sync_copy(idx_hbm.at[pl.ds(base, bpw)], idx_v)
        pltpu.async_copy(table_hbm.at[idx_v], rows_v, sem).wait()
        pltpu.sync_copy(rows_v, out_hbm.at[pl.ds(base, bpw)])

    return k(table, idx)


def _prod_to_flat_idx(prod, Bb):
    idx = (prod[..., 0] * 64.0 + prod[..., 1]).astype(jnp.int32)
    return (idx + (jnp.arange(Bb, dtype=jnp.int32) * NM)[:, None]).reshape(-1)


def kernel(memory, ref_points, params):
    f32 = jnp.float32
    Bb = memory.shape[0]

    base = {
        "memory": memory.astype(f32),
        "rpx_c": ref_points[:, :, 0:1].astype(f32),
        "rpy_c": ref_points[:, :, 1:2].astype(f32),
        "rpx_r": ref_points[:, :, 0:1].transpose(0, 2, 1).astype(f32),
        "rpy_r": ref_points[:, :, 1:2].transpose(0, 2, 1).astype(f32),
        "query_feat": params["query_feat"].astype(f32),
        "query_pos": params["query_pos"].astype(f32),
        "bato_q_proj": params["bato"]["q_proj"].astype(f32),
        "bato_m_proj": params["bato"]["m_proj"].astype(f32),
        "bato_g1_w": params["bato"]["g1_w"].astype(f32),
        "bato_g1_b": params["bato"]["g1_b"].reshape(1, -1).astype(f32),
        "bato_g2_w": params["bato"]["g2_w"].astype(f32),
        "bato_g2_b": params["bato"]["g2_b"].reshape(1).astype(f32),
        "pr_w1": params["posrel"]["w1"].astype(f32),
        "pr_b1": params["posrel"]["b1"].reshape(NPF).astype(f32),
        "pr_w2t": params["posrel"]["w2"].T.astype(f32),
        "pr_b2": params["posrel"]["b2"].reshape(NH).astype(f32),
    }
    for k in _LAYER_KEYS:
        stk = jnp.stack([params["layers"][l][k] for l in range(NL)])
        if stk.ndim == 2:
            stk = stk[:, None, :]
        base["L_" + k] = stk.astype(f32)

    _SMEM = ("pr_w1", "pr_b1", "pr_w2t", "pr_b2", "bato_g2_b")
    _BATCHED = ("memory", "rpx_c", "rpy_c", "rpx_r", "rpy_r",
                "q_in", "memb_in", "msel_in", "bias_in")

    def bspec(a, k):
        nd = a.ndim
        if k in _BATCHED:
            blk = (1,) + a.shape[1:]
            return pl.BlockSpec(blk, lambda b, _n=nd: (b,) + (0,) * (_n - 1))
        if k in _SMEM:
            return pl.BlockSpec(a.shape, lambda b, _n=nd: (0,) * _n,
                                memory_space=pltpu.SMEM)
        return pl.BlockSpec(a.shape, lambda b, _n=nd: (0,) * _n)

    def call(body, inputs, outs):
        names = list(inputs.keys())
        arrays = [inputs[k] for k in names]
        return pl.pallas_call(
            lambda *refs: body(names, *refs),
            grid=(Bb,),
            in_specs=[bspec(a, k) for k, a in zip(names, arrays)],
            out_specs=[pl.BlockSpec((1,) + s[1:],
                                    lambda b, _n=len(s): (b,) + (0,) * (_n - 1))
                       for s in outs],
            out_shape=[jax.ShapeDtypeStruct(s, f32) for s in outs],
        )(*arrays)

    def sub(d, drop_l=(), drop=()):
        out = {}
        for k, v in d.items():
            if k.startswith("L_") and k[2:] in drop_l:
                continue
            if k in drop:
                continue
            out[k] = v
        return out

    _CA_FFN = ("ca_in_w", "ca_in_b", "ca_out_w", "ca_out_b",
               "ffn1_w", "ffn1_b", "ffn2_w", "ffn2_b",
               "n2_g", "n2_b", "n3_g", "n3_b")
    _PRE = ("memory", "rpx_c", "rpy_c", "rpx_r", "rpy_r", "query_feat",
            "bato_q_proj", "bato_m_proj", "bato_g1_w", "bato_g1_b",
            "bato_g2_w", "bato_g2_b", "pr_w1", "pr_b1", "pr_w2t", "pr_b2")

    q1, memb, bias, prod0 = call(
        _stage1_kernel, sub(base, drop_l=_CA_FFN),
        [(Bb, NQ, D), (Bb, NM, D), (Bb, NH, NQ, NQ), (Bb, CK, 8)])

    table = memb.reshape(Bb * NM, D)
    msel0 = _sc_gather(table, _prod_to_flat_idx(prod0, Bb))

    s2 = sub(base, drop=_PRE)
    s2.update(q_in=q1, memb_in=memb, bias_in=bias,
              msel_in=msel0.reshape(Bb, CK, D))
    q2, prod1 = call(_stage2_kernel, s2, [(Bb, NQ, D), (Bb, CK, 8)])

    msel1 = _sc_gather(table, _prod_to_flat_idx(prod1, Bb))

    s3 = sub(base, drop=_PRE,
             drop_l=tuple(k for k in _LAYER_KEYS if k not in _CA_FFN))
    s3.update(q_in=q2, msel_in=msel1.reshape(Bb, CK, D))
    (out,) = call(_stage3_kernel, s3, [(Bb, NQ, D)])
    return out
